# Initial kernel scaffold; baseline (speedup 1.0000x reference)
import functools

import jax
import jax.numpy as jnp
from jax import lax
from jax.experimental import pallas as pl
from jax.experimental.pallas import tpu as pltpu
from jax.experimental.pallas import tpu_sc as plsc

N = 10000
E = 320000
D = 128
H = 128
C = 16

NC = 2
NS = 16
NW = NC * NS
CHUNK = 128
EPW = 10240
NCHUNK = EPW // CHUNK
EPAD = EPW * NW
NPAD = 10240
SLAB = NPAD // NS

_MESH = plsc.VectorSubcoreMesh(core_axis_name="c", subcore_axis_name="s")


def _deg_body(src_hbm, dst_hbm, zero_hbm, out_hbm, src_v, dst_v, ones_v,
              dego_s, degi_s):
    c = lax.axis_index("c")
    s = lax.axis_index("s")
    w = c * NS + s
    pltpu.sync_copy(zero_hbm.at[pl.ds(s * SLAB, SLAB)],
                    dego_s.at[pl.ds(s * SLAB, SLAB)])
    pltpu.sync_copy(zero_hbm.at[pl.ds(s * SLAB, SLAB)],
                    degi_s.at[pl.ds(s * SLAB, SLAB)])
    pltpu.sync_copy(src_hbm.at[w], src_v)
    pltpu.sync_copy(dst_hbm.at[w], dst_v)
    for k in range(CHUNK // 16):
        ones_v[pl.ds(k * 16, 16)] = jnp.ones((16,), jnp.float32)
    plsc.subcore_barrier()

    def body(j, carry):
        pltpu.sync_copy(ones_v, dego_s.at[src_v.at[j]], add=True)
        pltpu.sync_copy(ones_v, degi_s.at[dst_v.at[j]], add=True)
        return carry

    lax.fori_loop(0, NCHUNK, body, 0)
    plsc.subcore_barrier()
    pltpu.sync_copy(dego_s.at[pl.ds(s * SLAB, SLAB)],
                    out_hbm.at[c, 0, pl.ds(s * SLAB, SLAB)])
    pltpu.sync_copy(degi_s.at[pl.ds(s * SLAB, SLAB)],
                    out_hbm.at[c, 1, pl.ds(s * SLAB, SLAB)])


def _agg_body(hs_hbm, src_hbm, dst_hbm, zero_hbm, out_hbm, src_v, dst_v,
              bufa, bufb, acc_s, sema, semb):
    c = lax.axis_index("c")
    s = lax.axis_index("s")
    w = c * NS + s
    pltpu.sync_copy(zero_hbm.at[pl.ds(s * SLAB, SLAB)],
                    acc_s.at[pl.ds(s * SLAB, SLAB)])
    pltpu.sync_copy(src_hbm.at[w], src_v)
    pltpu.sync_copy(dst_hbm.at[w], dst_v)
    plsc.subcore_barrier()

    def body(i, carry):
        a = 2 * i
        b = 2 * i + 1
        da = pltpu.async_copy(hs_hbm.at[src_v.at[a]], bufa, sema)
        db = pltpu.async_copy(hs_hbm.at[src_v.at[b]], bufb, semb)
        da.wait()
        pltpu.sync_copy(bufa, acc_s.at[dst_v.at[a]], add=True)
        db.wait()
        pltpu.sync_copy(bufb, acc_s.at[dst_v.at[b]], add=True)
        return carry

    lax.fori_loop(0, NCHUNK // 2, body, 0)
    plsc.subcore_barrier()
    pltpu.sync_copy(acc_s.at[pl.ds(s * SLAB, SLAB)],
                    out_hbm.at[c, pl.ds(s * SLAB, SLAB)])


_deg_call = pl.kernel(
    _deg_body,
    out_type=jax.ShapeDtypeStruct((NC, 2, NPAD), jnp.float32),
    mesh=_MESH,
    scratch_types=[
        pltpu.VMEM((NCHUNK, CHUNK), jnp.int32),
        pltpu.VMEM((NCHUNK, CHUNK), jnp.int32),
        pltpu.VMEM((CHUNK,), jnp.float32),
        pltpu.VMEM_SHARED((NPAD,), jnp.float32),
        pltpu.VMEM_SHARED((NPAD,), jnp.float32),
    ],
)

_agg_call = pl.kernel(
    _agg_body,
    out_type=jax.ShapeDtypeStruct((NC, NPAD, H), jnp.float32),
    mesh=_MESH,
    scratch_types=[
        pltpu.VMEM((NCHUNK, CHUNK), jnp.int32),
        pltpu.VMEM((NCHUNK, CHUNK), jnp.int32),
        pltpu.VMEM((CHUNK, H), jnp.float32),
        pltpu.VMEM((CHUNK, H), jnp.float32),
        pltpu.VMEM_SHARED((NPAD, H), jnp.float32),
        pltpu.SemaphoreType.DMA,
        pltpu.SemaphoreType.DMA,
    ],
)


def _norms(degp):
    normo = lax.rsqrt(jnp.maximum(degp[0, 0] + degp[1, 0], 1.0))
    normi = lax.rsqrt(jnp.maximum(degp[0, 1] + degp[1, 1], 1.0))
    return normo, normi


def _fc_body(x_ref, w_ref, b_ref, degp_ref, out_ref):
    normo, _ = _norms(degp_ref[...])
    h = jnp.dot(x_ref[...], w_ref[...], preferred_element_type=jnp.float32)
    out_ref[...] = (h + b_ref[...]) * normo


def _layer_a_body(p_ref, degp_ref, b_ref, out_ref):
    normo, normi = _norms(degp_ref[...])
    agg = (p_ref[0] + p_ref[1]) * normi
    h = jnp.maximum(agg + b_ref[...], 0.0)
    out_ref[...] = h * normo


def _layer_b_body(p_ref, degp_ref, w_ref, b_ref, h_ref, hs_ref):
    normo, normi = _norms(degp_ref[...])
    agg = (p_ref[0] + p_ref[1]) * normi
    h = jnp.dot(agg, w_ref[...], preferred_element_type=jnp.float32)
    h = jnp.maximum(h + b_ref[...], 0.0)
    h_ref[...] = h
    hs_ref[...] = h * normo


def _layer_c_body(p_ref, degp_ref, w_ref, b_ref, out_ref):
    _, normi = _norms(degp_ref[...])
    agg = (p_ref[0] + p_ref[1]) * normi
    h = jnp.dot(agg, w_ref[...], preferred_element_type=jnp.float32)
    out_ref[...] = h + b_ref[...]


_GRID = (N + 127) // 128

_degp_spec = pl.BlockSpec((NC, 2, 128, 1), lambda i: (0, 0, i, 0))
_row_spec = pl.BlockSpec((128, H), lambda i: (i, 0))
_p_spec = pl.BlockSpec((NC, 128, H), lambda i: (0, i, 0))


def _mk_fc():
    return pl.pallas_call(
        _fc_body,
        grid=(_GRID,),
        in_specs=[_row_spec,
                  pl.BlockSpec((D, H), lambda i: (0, 0)),
                  pl.BlockSpec((1, H), lambda i: (0, 0)),
                  _degp_spec],
        out_specs=_row_spec,
        out_shape=jax.ShapeDtypeStruct((N, H), jnp.float32),
    )


def _mk_layer_a():
    return pl.pallas_call(
        _layer_a_body,
        grid=(_GRID,),
        in_specs=[_p_spec, _degp_spec,
                  pl.BlockSpec((1, H), lambda i: (0, 0))],
        out_specs=_row_spec,
        out_shape=jax.ShapeDtypeStruct((N, H), jnp.float32),
    )


def _mk_layer_b():
    return pl.pallas_call(
        _layer_b_body,
        grid=(_GRID,),
        in_specs=[_p_spec, _degp_spec,
                  pl.BlockSpec((H, H), lambda i: (0, 0)),
                  pl.BlockSpec((1, H), lambda i: (0, 0))],
        out_specs=[_row_spec, _row_spec],
        out_shape=[jax.ShapeDtypeStruct((N, H), jnp.float32),
                   jax.ShapeDtypeStruct((N, H), jnp.float32)],
    )


def _mk_layer_c():
    return pl.pallas_call(
        _layer_c_body,
        grid=(_GRID,),
        in_specs=[_p_spec, _degp_spec,
                  pl.BlockSpec((H, C), lambda i: (0, 0)),
                  pl.BlockSpec((1, C), lambda i: (0, 0))],
        out_specs=pl.BlockSpec((128, C), lambda i: (i, 0)),
        out_shape=jax.ShapeDtypeStruct((N, C), jnp.float32),
    )


def kernel(features_list, edge_index, e_feat, W_fc, b_fc, b0, W1, b1, W2, b2):
    src = edge_index[0]
    dst = edge_index[1]
    pad = EPAD - E
    src_g = jnp.concatenate([src, jnp.zeros((pad,), jnp.int32)])
    src_d = jnp.concatenate([src, jnp.full((pad,), N, jnp.int32)])
    dst_p = jnp.concatenate([dst, jnp.full((pad,), N, jnp.int32)])
    src_g3 = src_g.reshape(NW, NCHUNK, CHUNK)
    src_d3 = src_d.reshape(NW, NCHUNK, CHUNK)
    dst_p3 = dst_p.reshape(NW, NCHUNK, CHUNK)

    zero1 = jnp.zeros((NPAD,), jnp.float32)
    zero2 = jnp.zeros((NPAD, H), jnp.float32)

    degp = _deg_call(src_d3, dst_p3, zero1)
    degp4 = degp.reshape(NC, 2, NPAD, 1)

    b_fc2 = b_fc.reshape(1, H)
    b0_2 = b0.reshape(1, H)
    b1_2 = b1.reshape(1, H)
    b2_2 = b2.reshape(1, C)

    hs0 = _mk_fc()(features_list, W_fc, b_fc2, degp4)
    p1 = _agg_call(hs0, src_g3, dst_p3, zero2)
    hs1 = _mk_layer_a()(p1, degp4, b0_2)
    p2 = _agg_call(hs1, src_g3, dst_p3, zero2)
    h2, hs2 = _mk_layer_b()(p2, degp4, W1, b1_2)
    p3 = _agg_call(hs2, src_g3, dst_p3, zero2)
    out = _mk_layer_c()(p3, degp4, W2, b2_2)
    return (out, h2)


# speedup vs baseline: 2.7696x; 2.7696x over previous
import functools

import jax
import jax.numpy as jnp
from jax import lax
from jax.experimental import pallas as pl
from jax.experimental.pallas import tpu as pltpu
from jax.experimental.pallas import tpu_sc as plsc

N = 10000
E = 320000
D = 128
H = 128
C = 16

NC = 2
NS = 16
NW = NC * NS
CHUNK = 128
EPW = 10240
NCHUNK = EPW // CHUNK
EPAD = EPW * NW
NPAD = 10240
SLAB = NPAD // NS

_MESH = plsc.VectorSubcoreMesh(core_axis_name="c", subcore_axis_name="s")


def _unpack_indices(src_v, dst_v, nchunk):
    def body(j, carry):
        for k in range(CHUNK // 16):
            p = src_v[j, pl.ds(k * 16, 16)]
            src_v[j, pl.ds(k * 16, 16)] = lax.shift_right_logical(p, 14)
            dst_v[j, pl.ds(k * 16, 16)] = lax.bitwise_and(p, 16383)
        return carry

    lax.fori_loop(0, nchunk, body, 0)


def _deg_body(pk_hbm, zero_hbm, out_hbm, src_v, dst_v, ones_v,
              dego_s, degi_s):
    c = lax.axis_index("c")
    s = lax.axis_index("s")
    w = c * NS + s
    pltpu.sync_copy(zero_hbm.at[pl.ds(s * SLAB, SLAB)],
                    dego_s.at[pl.ds(s * SLAB, SLAB)])
    pltpu.sync_copy(zero_hbm.at[pl.ds(s * SLAB, SLAB)],
                    degi_s.at[pl.ds(s * SLAB, SLAB)])
    pltpu.sync_copy(pk_hbm.at[w], src_v)
    _unpack_indices(src_v, dst_v, NCHUNK)
    for k in range(CHUNK // 16):
        ones_v[pl.ds(k * 16, 16)] = jnp.ones((16,), jnp.float32)
    plsc.subcore_barrier()

    def body(j, carry):
        pltpu.sync_copy(ones_v, dego_s.at[src_v.at[j]], add=True)
        pltpu.sync_copy(ones_v, degi_s.at[dst_v.at[j]], add=True)
        return carry

    lax.fori_loop(0, NCHUNK, body, 0)
    plsc.subcore_barrier()
    pltpu.sync_copy(dego_s.at[pl.ds(s * SLAB, SLAB)],
                    out_hbm.at[c, 0, pl.ds(s * SLAB, SLAB)])
    pltpu.sync_copy(degi_s.at[pl.ds(s * SLAB, SLAB)],
                    out_hbm.at[c, 1, pl.ds(s * SLAB, SLAB)])


NHALF = NCHUNK // 2


def _agg_body(hs_hbm, pk_hbm, zero_hbm, out_hbm, src_v, dst_v,
              bufa, bufb, acc_s, sema, semb):
    c = lax.axis_index("c")
    s = lax.axis_index("s")
    w = c * NS + s
    pltpu.sync_copy(zero_hbm.at[pl.ds(s * SLAB, SLAB)],
                    acc_s.at[pl.ds(s * SLAB, SLAB)])
    plsc.subcore_barrier()

    def run_half(h0):
        pltpu.sync_copy(pk_hbm.at[w, pl.ds(h0, NHALF)], src_v)
        _unpack_indices(src_v, dst_v, NHALF)

        def body(i, carry):
            a = 2 * i
            b = 2 * i + 1
            da = pltpu.async_copy(hs_hbm.at[src_v.at[a]], bufa, sema)
            db = pltpu.async_copy(hs_hbm.at[src_v.at[b]], bufb, semb)
            da.wait()
            pltpu.sync_copy(bufa, acc_s.at[dst_v.at[a]], add=True)
            db.wait()
            pltpu.sync_copy(bufb, acc_s.at[dst_v.at[b]], add=True)
            return carry

        lax.fori_loop(0, NHALF // 2, body, 0)

    run_half(0)
    run_half(NHALF)
    plsc.subcore_barrier()
    pltpu.sync_copy(acc_s.at[pl.ds(s * SLAB, SLAB)],
                    out_hbm.at[c, pl.ds(s * SLAB, SLAB)])


_deg_call = pl.kernel(
    _deg_body,
    out_type=jax.ShapeDtypeStruct((NC, 2, NPAD), jnp.float32),
    mesh=_MESH,
    scratch_types=[
        pltpu.VMEM((NCHUNK, CHUNK), jnp.int32),
        pltpu.VMEM((NCHUNK, CHUNK), jnp.int32),
        pltpu.VMEM((CHUNK,), jnp.float32),
        pltpu.VMEM_SHARED((NPAD,), jnp.float32),
        pltpu.VMEM_SHARED((NPAD,), jnp.float32),
    ],
)

_agg_call = pl.kernel(
    _agg_body,
    out_type=jax.ShapeDtypeStruct((NC, NPAD, H), jnp.float32),
    mesh=_MESH,
    scratch_types=[
        pltpu.VMEM((NCHUNK // 2, CHUNK), jnp.int32),
        pltpu.VMEM((NCHUNK // 2, CHUNK), jnp.int32),
        pltpu.VMEM((CHUNK, H), jnp.float32),
        pltpu.VMEM((CHUNK, H), jnp.float32),
        pltpu.VMEM_SHARED((NPAD, H), jnp.float32),
        pltpu.SemaphoreType.DMA,
        pltpu.SemaphoreType.DMA,
    ],
)


def _norms(degp):
    normo = lax.rsqrt(jnp.maximum(degp[0, 0] + degp[1, 0], 1.0))
    normi = lax.rsqrt(jnp.maximum(degp[0, 1] + degp[1, 1], 1.0))
    return normo, normi


def _fc_body(x_ref, w_ref, b_ref, degp_ref, out_ref):
    normo, _ = _norms(degp_ref[...])
    h = jnp.dot(x_ref[...], w_ref[...], preferred_element_type=jnp.float32)
    out_ref[...] = (h + b_ref[...]) * normo


def _layer_a_body(p_ref, degp_ref, b_ref, out_ref):
    normo, normi = _norms(degp_ref[...])
    agg = (p_ref[0] + p_ref[1]) * normi
    h = jnp.maximum(agg + b_ref[...], 0.0)
    out_ref[...] = h * normo


def _layer_b_body(p_ref, degp_ref, w_ref, b_ref, h_ref, hs_ref):
    normo, normi = _norms(degp_ref[...])
    agg = (p_ref[0] + p_ref[1]) * normi
    h = jnp.dot(agg, w_ref[...], preferred_element_type=jnp.float32)
    h = jnp.maximum(h + b_ref[...], 0.0)
    h_ref[...] = h
    hs_ref[...] = h * normo


def _layer_c_body(p_ref, degp_ref, w_ref, b_ref, out_ref):
    _, normi = _norms(degp_ref[...])
    agg = (p_ref[0] + p_ref[1]) * normi
    h = jnp.dot(agg, w_ref[...], preferred_element_type=jnp.float32)
    out_ref[...] = h + b_ref[...]


_GRID = (N + 127) // 128

_degp_spec = pl.BlockSpec((NC, 2, 128, 1), lambda i: (0, 0, i, 0))
_row_spec = pl.BlockSpec((128, H), lambda i: (i, 0))
_p_spec = pl.BlockSpec((NC, 128, H), lambda i: (0, i, 0))


def _mk_fc():
    return pl.pallas_call(
        _fc_body,
        grid=(_GRID,),
        in_specs=[_row_spec,
                  pl.BlockSpec((D, H), lambda i: (0, 0)),
                  pl.BlockSpec((1, H), lambda i: (0, 0)),
                  _degp_spec],
        out_specs=_row_spec,
        out_shape=jax.ShapeDtypeStruct((N, H), jnp.float32),
    )


def _mk_layer_a():
    return pl.pallas_call(
        _layer_a_body,
        grid=(_GRID,),
        in_specs=[_p_spec, _degp_spec,
                  pl.BlockSpec((1, H), lambda i: (0, 0))],
        out_specs=_row_spec,
        out_shape=jax.ShapeDtypeStruct((N, H), jnp.float32),
    )


def _mk_layer_b():
    return pl.pallas_call(
        _layer_b_body,
        grid=(_GRID,),
        in_specs=[_p_spec, _degp_spec,
                  pl.BlockSpec((H, H), lambda i: (0, 0)),
                  pl.BlockSpec((1, H), lambda i: (0, 0))],
        out_specs=[_row_spec, _row_spec],
        out_shape=[jax.ShapeDtypeStruct((N, H), jnp.float32),
                   jax.ShapeDtypeStruct((N, H), jnp.float32)],
    )


def _mk_layer_c():
    return pl.pallas_call(
        _layer_c_body,
        grid=(_GRID,),
        in_specs=[_p_spec, _degp_spec,
                  pl.BlockSpec((H, C), lambda i: (0, 0)),
                  pl.BlockSpec((1, C), lambda i: (0, 0))],
        out_specs=pl.BlockSpec((128, C), lambda i: (i, 0)),
        out_shape=jax.ShapeDtypeStruct((N, C), jnp.float32),
    )


def kernel(features_list, edge_index, e_feat, W_fc, b_fc, b0, W1, b1, W2, b2):
    src = edge_index[0]
    dst = edge_index[1]
    pad = EPAD - E
    src_g = jnp.concatenate([src, jnp.zeros((pad,), jnp.int32)])
    src_d = jnp.concatenate([src, jnp.full((pad,), N, jnp.int32)])
    dst_p = jnp.concatenate([dst, jnp.full((pad,), N, jnp.int32)])
    pk_g3 = ((src_g << 14) | dst_p).reshape(NW, NCHUNK, CHUNK)
    pk_d3 = ((src_d << 14) | dst_p).reshape(NW, NCHUNK, CHUNK)

    zero1 = jnp.zeros((NPAD,), jnp.float32)
    zero2 = jnp.zeros((NPAD, H), jnp.float32)

    degp = _deg_call(pk_d3, zero1)
    degp4 = degp.reshape(NC, 2, NPAD, 1)

    b_fc2 = b_fc.reshape(1, H)
    b0_2 = b0.reshape(1, H)
    b1_2 = b1.reshape(1, H)
    b2_2 = b2.reshape(1, C)

    hs0 = _mk_fc()(features_list, W_fc, b_fc2, degp4)
    p1 = _agg_call(hs0, pk_g3, zero2)
    hs1 = _mk_layer_a()(p1, degp4, b0_2)
    p2 = _agg_call(hs1, pk_g3, zero2)
    h2, hs2 = _mk_layer_b()(p2, degp4, W1, b1_2)
    p3 = _agg_call(hs2, pk_g3, zero2)
    out = _mk_layer_c()(p3, degp4, W2, b2_2)
    return (out, h2)


# speedup vs baseline: 6.8638x; 2.4783x over previous
import functools

import jax
import jax.numpy as jnp
from jax import lax
from jax.experimental import pallas as pl
from jax.experimental.pallas import tpu as pltpu
from jax.experimental.pallas import tpu_sc as plsc

N = 10000
E = 320000
D = 128
H = 128
C = 16

NC = 2
NS = 16
NW = NC * NS
CHUNK = 128
EPW = 10240
NCHUNK = EPW // CHUNK
EPAD = EPW * NW
NPAD = 10240
SLAB = NPAD // NS

_MESH = plsc.VectorSubcoreMesh(core_axis_name="c", subcore_axis_name="s")


def _unpack_indices(src_v, dst_v, nchunk):
    def body(j, carry):
        for k in range(CHUNK // 16):
            p = src_v[j, pl.ds(k * 16, 16)]
            src_v[j, pl.ds(k * 16, 16)] = lax.shift_right_logical(p, 14)
            dst_v[j, pl.ds(k * 16, 16)] = lax.bitwise_and(p, 16383)
        return carry

    lax.fori_loop(0, nchunk, body, 0)


def _deg_body(pk_hbm, zero_hbm, out_hbm, src_v, dst_v, ones_v,
              dego_s, degi_s):
    c = lax.axis_index("c")
    s = lax.axis_index("s")
    w = c * NS + s
    pltpu.sync_copy(zero_hbm.at[pl.ds(s * SLAB, SLAB)],
                    dego_s.at[pl.ds(s * SLAB, SLAB)])
    pltpu.sync_copy(zero_hbm.at[pl.ds(s * SLAB, SLAB)],
                    degi_s.at[pl.ds(s * SLAB, SLAB)])
    pltpu.sync_copy(pk_hbm.at[w], src_v)
    _unpack_indices(src_v, dst_v, NCHUNK)
    for k in range(CHUNK // 16):
        ones_v[pl.ds(k * 16, 16)] = jnp.ones((16,), jnp.float32)
    plsc.subcore_barrier()

    def body(j, carry):
        pltpu.sync_copy(ones_v, dego_s.at[src_v.at[j]], add=True)
        pltpu.sync_copy(ones_v, degi_s.at[dst_v.at[j]], add=True)
        return carry

    lax.fori_loop(0, NCHUNK, body, 0)
    plsc.subcore_barrier()
    pltpu.sync_copy(dego_s.at[pl.ds(s * SLAB, SLAB)],
                    out_hbm.at[c, 0, pl.ds(s * SLAB, SLAB)])
    pltpu.sync_copy(degi_s.at[pl.ds(s * SLAB, SLAB)],
                    out_hbm.at[c, 1, pl.ds(s * SLAB, SLAB)])


NHALF = NCHUNK // 2


def _agg_body(hs_hbm, pk_hbm, zero_hbm, out_hbm, src_v, dst_v,
              bufa, bufb, acc_s, sema, semb):
    c = lax.axis_index("c")
    s = lax.axis_index("s")
    w = c * NS + s
    pltpu.sync_copy(zero_hbm.at[pl.ds(s * SLAB, SLAB)],
                    acc_s.at[pl.ds(s * SLAB, SLAB)])
    plsc.subcore_barrier()

    def run_half(h0):
        pltpu.sync_copy(pk_hbm.at[w, pl.ds(h0, NHALF)], src_v)
        _unpack_indices(src_v, dst_v, NHALF)

        def body(i, carry):
            a = 2 * i
            b = 2 * i + 1
            da = pltpu.async_copy(hs_hbm.at[src_v.at[a]], bufa, sema)
            db = pltpu.async_copy(hs_hbm.at[src_v.at[b]], bufb, semb)
            da.wait()
            pltpu.sync_copy(bufa, acc_s.at[dst_v.at[a]], add=True)
            db.wait()
            pltpu.sync_copy(bufb, acc_s.at[dst_v.at[b]], add=True)
            return carry

        lax.fori_loop(0, NHALF // 2, body, 0)

    run_half(0)
    run_half(NHALF)
    plsc.subcore_barrier()
    pltpu.sync_copy(acc_s.at[pl.ds(s * SLAB, SLAB)],
                    out_hbm.at[c, pl.ds(s * SLAB, SLAB)])


_deg_call = pl.kernel(
    _deg_body,
    out_type=jax.ShapeDtypeStruct((NC, 2, NPAD), jnp.float32),
    mesh=_MESH,
    scratch_types=[
        pltpu.VMEM((NCHUNK, CHUNK), jnp.int32),
        pltpu.VMEM((NCHUNK, CHUNK), jnp.int32),
        pltpu.VMEM((CHUNK,), jnp.float32),
        pltpu.VMEM_SHARED((NPAD,), jnp.float32),
        pltpu.VMEM_SHARED((NPAD,), jnp.float32),
    ],
)

_agg_call = pl.kernel(
    _agg_body,
    out_type=jax.ShapeDtypeStruct((NC, NPAD, H), jnp.float32),
    mesh=_MESH,
    scratch_types=[
        pltpu.VMEM((NCHUNK // 2, CHUNK), jnp.int32),
        pltpu.VMEM((NCHUNK // 2, CHUNK), jnp.int32),
        pltpu.VMEM((CHUNK, H), jnp.float32),
        pltpu.VMEM((CHUNK, H), jnp.float32),
        pltpu.VMEM_SHARED((NPAD, H), jnp.float32),
        pltpu.SemaphoreType.DMA,
        pltpu.SemaphoreType.DMA,
    ],
)


def _norms(degp):
    normo = lax.rsqrt(jnp.maximum(degp[0, 0] + degp[1, 0], 1.0))
    normi = lax.rsqrt(jnp.maximum(degp[0, 1] + degp[1, 1], 1.0))
    return normo, normi


def _fc_body(x_ref, w_ref, b_ref, degp_ref, out_ref):
    normo, _ = _norms(degp_ref[...])
    h = jnp.dot(x_ref[...], w_ref[...], preferred_element_type=jnp.float32)
    out_ref[...] = (h + b_ref[...]) * normo


def _layer_a_body(p_ref, degp_ref, b_ref, out_ref):
    normo, normi = _norms(degp_ref[...])
    agg = (p_ref[0] + p_ref[1]) * normi
    h = jnp.maximum(agg + b_ref[...], 0.0)
    out_ref[...] = h * normo


def _layer_b_body(p_ref, degp_ref, w_ref, b_ref, h_ref, hs_ref):
    normo, normi = _norms(degp_ref[...])
    agg = (p_ref[0] + p_ref[1]) * normi
    h = jnp.dot(agg, w_ref[...], preferred_element_type=jnp.float32)
    h = jnp.maximum(h + b_ref[...], 0.0)
    h_ref[...] = h
    hs_ref[...] = h * normo


def _layer_c_body(p_ref, degp_ref, w_ref, b_ref, out_ref):
    _, normi = _norms(degp_ref[...])
    agg = (p_ref[0] + p_ref[1]) * normi
    h = jnp.dot(agg, w_ref[...], preferred_element_type=jnp.float32)
    out_ref[...] = h + b_ref[...]


_GRID = (N + 127) // 128

_degp_spec = pl.BlockSpec((NC, 2, 128, 1), lambda i: (0, 0, i, 0))
_row_spec = pl.BlockSpec((128, H), lambda i: (i, 0))
_p_spec = pl.BlockSpec((NC, 128, H), lambda i: (0, i, 0))


def _mk_fc():
    return pl.pallas_call(
        _fc_body,
        grid=(_GRID,),
        in_specs=[_row_spec,
                  pl.BlockSpec((D, H), lambda i: (0, 0)),
                  pl.BlockSpec((1, H), lambda i: (0, 0)),
                  _degp_spec],
        out_specs=_row_spec,
        out_shape=jax.ShapeDtypeStruct((N, H), jnp.float32),
    )


def _mk_layer_a():
    return pl.pallas_call(
        _layer_a_body,
        grid=(_GRID,),
        in_specs=[_p_spec, _degp_spec,
                  pl.BlockSpec((1, H), lambda i: (0, 0))],
        out_specs=_row_spec,
        out_shape=jax.ShapeDtypeStruct((N, H), jnp.float32),
    )


def _mk_layer_b():
    return pl.pallas_call(
        _layer_b_body,
        grid=(_GRID,),
        in_specs=[_p_spec, _degp_spec,
                  pl.BlockSpec((H, H), lambda i: (0, 0)),
                  pl.BlockSpec((1, H), lambda i: (0, 0))],
        out_specs=[_row_spec, _row_spec],
        out_shape=[jax.ShapeDtypeStruct((N, H), jnp.float32),
                   jax.ShapeDtypeStruct((N, H), jnp.float32)],
    )


def _mk_layer_c():
    return pl.pallas_call(
        _layer_c_body,
        grid=(_GRID,),
        in_specs=[_p_spec, _degp_spec,
                  pl.BlockSpec((H, C), lambda i: (0, 0)),
                  pl.BlockSpec((1, C), lambda i: (0, 0))],
        out_specs=pl.BlockSpec((128, C), lambda i: (i, 0)),
        out_shape=jax.ShapeDtypeStruct((N, C), jnp.float32),
    )


def kernel(features_list, edge_index, e_feat, W_fc, b_fc, b0, W1, b1, W2, b2):
    src = edge_index[0]
    dst = edge_index[1]
    padw = EPW - E // NW
    tr = jnp.arange(padw, dtype=jnp.int32)
    t_dst = jnp.broadcast_to(N + tr % (NPAD - N), (NW, padw))
    t_src_g = jnp.broadcast_to(tr % N, (NW, padw))
    t_src_d = t_dst
    src_g = jnp.concatenate([src.reshape(NW, E // NW), t_src_g], axis=1)
    src_d = jnp.concatenate([src.reshape(NW, E // NW), t_src_d], axis=1)
    dst_p = jnp.concatenate([dst.reshape(NW, E // NW), t_dst], axis=1)
    pk_g3 = ((src_g << 14) | dst_p).reshape(NW, NCHUNK, CHUNK)
    pk_d3 = ((src_d << 14) | dst_p).reshape(NW, NCHUNK, CHUNK)

    zero1 = jnp.zeros((NPAD,), jnp.float32)
    zero2 = jnp.zeros((NPAD, H), jnp.float32)

    degp = _deg_call(pk_d3, zero1)
    degp4 = degp.reshape(NC, 2, NPAD, 1)

    b_fc2 = b_fc.reshape(1, H)
    b0_2 = b0.reshape(1, H)
    b1_2 = b1.reshape(1, H)
    b2_2 = b2.reshape(1, C)

    hs0 = _mk_fc()(features_list, W_fc, b_fc2, degp4)
    p1 = _agg_call(hs0, pk_g3, zero2)
    hs1 = _mk_layer_a()(p1, degp4, b0_2)
    p2 = _agg_call(hs1, pk_g3, zero2)
    h2, hs2 = _mk_layer_b()(p2, degp4, W1, b1_2)
    p3 = _agg_call(hs2, pk_g3, zero2)
    out = _mk_layer_c()(p3, degp4, W2, b2_2)
    return (out, h2)


# speedup vs baseline: 7.4842x; 1.0904x over previous
import functools

import jax
import jax.numpy as jnp
from jax import lax
from jax.experimental import pallas as pl
from jax.experimental.pallas import tpu as pltpu
from jax.experimental.pallas import tpu_sc as plsc

N = 10000
E = 320000
D = 128
H = 128
C = 16

NC = 2
NS = 16
NW = NC * NS
CHUNK = 128
EPW = 10240
NCHUNK = EPW // CHUNK
EPAD = EPW * NW
NPAD = 10240
SLAB = NPAD // NS

_MESH = plsc.VectorSubcoreMesh(core_axis_name="c", subcore_axis_name="s")


def _unpack_indices(src_v, dst_v, nchunk):
    def body(j, carry):
        for k in range(CHUNK // 16):
            p = src_v[j, pl.ds(k * 16, 16)]
            src_v[j, pl.ds(k * 16, 16)] = lax.shift_right_logical(p, 14)
            dst_v[j, pl.ds(k * 16, 16)] = lax.bitwise_and(p, 16383)
        return carry

    lax.fori_loop(0, nchunk, body, 0)


def _deg_body(pk_hbm, zero_hbm, out_hbm, src_v, dst_v, ones_v,
              dego_s, degi_s):
    c = lax.axis_index("c")
    s = lax.axis_index("s")
    w = c * NS + s
    pltpu.sync_copy(zero_hbm.at[pl.ds(s * SLAB, SLAB)],
                    dego_s.at[pl.ds(s * SLAB, SLAB)])
    pltpu.sync_copy(zero_hbm.at[pl.ds(s * SLAB, SLAB)],
                    degi_s.at[pl.ds(s * SLAB, SLAB)])
    pltpu.sync_copy(pk_hbm.at[w], src_v)
    _unpack_indices(src_v, dst_v, NCHUNK)
    for k in range(CHUNK // 16):
        ones_v[pl.ds(k * 16, 16)] = jnp.ones((16,), jnp.float32)
    plsc.subcore_barrier()

    def body(j, carry):
        pltpu.sync_copy(ones_v, dego_s.at[src_v.at[j]], add=True)
        pltpu.sync_copy(ones_v, degi_s.at[dst_v.at[j]], add=True)
        return carry

    lax.fori_loop(0, NCHUNK, body, 0)
    plsc.subcore_barrier()
    pltpu.sync_copy(dego_s.at[pl.ds(s * SLAB, SLAB)],
                    out_hbm.at[c, 0, pl.ds(s * SLAB, SLAB)])
    pltpu.sync_copy(degi_s.at[pl.ds(s * SLAB, SLAB)],
                    out_hbm.at[c, 1, pl.ds(s * SLAB, SLAB)])


NHALF = NCHUNK // 2


def _agg_body(hs_hbm, pk_hbm, zero_hbm, out_hbm, src_v, dst_v,
              bufa, bufb, acc_s, sga, sgb, ssa, ssb):
    c = lax.axis_index("c")
    s = lax.axis_index("s")
    w = c * NS + s
    pltpu.sync_copy(zero_hbm.at[pl.ds(s * SLAB, SLAB)],
                    acc_s.at[pl.ds(s * SLAB, SLAB)])
    plsc.subcore_barrier()

    def run_half(h0):
        pltpu.sync_copy(pk_hbm.at[w, pl.ds(h0, NHALF)], src_v)
        _unpack_indices(src_v, dst_v, NHALF)

        def g_start(j, buf, sem):
            pltpu.async_copy(hs_hbm.at[src_v.at[j]], buf, sem)

        def g_wait(j, buf, sem):
            pltpu.make_async_copy(hs_hbm.at[src_v.at[j]], buf, sem).wait()

        def s_start(j, buf, sem):
            pltpu.async_copy(buf, acc_s.at[dst_v.at[j]], sem, add=True)

        def s_wait(j, buf, sem):
            pltpu.make_async_copy(buf, acc_s.at[dst_v.at[j]], sem).wait()

        nit = NHALF // 2
        g_start(0, bufa, sga)

        def body(i, carry):
            a = 2 * i
            b = a + 1
            g_wait(a, bufa, sga)
            s_start(a, bufa, ssa)

            @pl.when(i > 0)
            def _():
                s_wait(b - 2, bufb, ssb)

            g_start(b, bufb, sgb)
            g_wait(b, bufb, sgb)
            s_start(b, bufb, ssb)
            s_wait(a, bufa, ssa)

            @pl.when(i < nit - 1)
            def _():
                g_start(a + 2, bufa, sga)

            return carry

        lax.fori_loop(0, nit, body, 0)
        s_wait(NHALF - 1, bufb, ssb)

    run_half(0)
    run_half(NHALF)
    plsc.subcore_barrier()
    pltpu.sync_copy(acc_s.at[pl.ds(s * SLAB, SLAB)],
                    out_hbm.at[c, pl.ds(s * SLAB, SLAB)])


_deg_call = pl.kernel(
    _deg_body,
    out_type=jax.ShapeDtypeStruct((NC, 2, NPAD), jnp.float32),
    mesh=_MESH,
    scratch_types=[
        pltpu.VMEM((NCHUNK, CHUNK), jnp.int32),
        pltpu.VMEM((NCHUNK, CHUNK), jnp.int32),
        pltpu.VMEM((CHUNK,), jnp.float32),
        pltpu.VMEM_SHARED((NPAD,), jnp.float32),
        pltpu.VMEM_SHARED((NPAD,), jnp.float32),
    ],
)

_agg_call = pl.kernel(
    _agg_body,
    out_type=jax.ShapeDtypeStruct((NC, NPAD, H), jnp.float32),
    mesh=_MESH,
    scratch_types=[
        pltpu.VMEM((NCHUNK // 2, CHUNK), jnp.int32),
        pltpu.VMEM((NCHUNK // 2, CHUNK), jnp.int32),
        pltpu.VMEM((CHUNK, H), jnp.float32),
        pltpu.VMEM((CHUNK, H), jnp.float32),
        pltpu.VMEM_SHARED((NPAD, H), jnp.float32),
        pltpu.SemaphoreType.DMA,
        pltpu.SemaphoreType.DMA,
        pltpu.SemaphoreType.DMA,
        pltpu.SemaphoreType.DMA,
    ],
)


def _norms(degp):
    normo = lax.rsqrt(jnp.maximum(degp[0, 0] + degp[1, 0], 1.0))
    normi = lax.rsqrt(jnp.maximum(degp[0, 1] + degp[1, 1], 1.0))
    return normo, normi


def _fc_body(x_ref, w_ref, b_ref, degp_ref, out_ref):
    normo, _ = _norms(degp_ref[...])
    h = jnp.dot(x_ref[...], w_ref[...], preferred_element_type=jnp.float32)
    out_ref[...] = (h + b_ref[...]) * normo


def _layer_a_body(p_ref, degp_ref, b_ref, out_ref):
    normo, normi = _norms(degp_ref[...])
    agg = (p_ref[0] + p_ref[1]) * normi
    h = jnp.maximum(agg + b_ref[...], 0.0)
    out_ref[...] = h * normo


def _layer_b_body(p_ref, degp_ref, w_ref, b_ref, h_ref, hs_ref):
    normo, normi = _norms(degp_ref[...])
    agg = (p_ref[0] + p_ref[1]) * normi
    h = jnp.dot(agg, w_ref[...], preferred_element_type=jnp.float32)
    h = jnp.maximum(h + b_ref[...], 0.0)
    h_ref[...] = h
    hs_ref[...] = h * normo


def _layer_c_body(p_ref, degp_ref, w_ref, b_ref, out_ref):
    _, normi = _norms(degp_ref[...])
    agg = (p_ref[0] + p_ref[1]) * normi
    h = jnp.dot(agg, w_ref[...], preferred_element_type=jnp.float32)
    out_ref[...] = h + b_ref[...]


_GRID = (N + 127) // 128

_degp_spec = pl.BlockSpec((NC, 2, 128, 1), lambda i: (0, 0, i, 0))
_row_spec = pl.BlockSpec((128, H), lambda i: (i, 0))
_p_spec = pl.BlockSpec((NC, 128, H), lambda i: (0, i, 0))


def _mk_fc():
    return pl.pallas_call(
        _fc_body,
        grid=(_GRID,),
        in_specs=[_row_spec,
                  pl.BlockSpec((D, H), lambda i: (0, 0)),
                  pl.BlockSpec((1, H), lambda i: (0, 0)),
                  _degp_spec],
        out_specs=_row_spec,
        out_shape=jax.ShapeDtypeStruct((N, H), jnp.float32),
    )


def _mk_layer_a():
    return pl.pallas_call(
        _layer_a_body,
        grid=(_GRID,),
        in_specs=[_p_spec, _degp_spec,
                  pl.BlockSpec((1, H), lambda i: (0, 0))],
        out_specs=_row_spec,
        out_shape=jax.ShapeDtypeStruct((N, H), jnp.float32),
    )


def _mk_layer_b():
    return pl.pallas_call(
        _layer_b_body,
        grid=(_GRID,),
        in_specs=[_p_spec, _degp_spec,
                  pl.BlockSpec((H, H), lambda i: (0, 0)),
                  pl.BlockSpec((1, H), lambda i: (0, 0))],
        out_specs=[_row_spec, _row_spec],
        out_shape=[jax.ShapeDtypeStruct((N, H), jnp.float32),
                   jax.ShapeDtypeStruct((N, H), jnp.float32)],
    )


def _mk_layer_c():
    return pl.pallas_call(
        _layer_c_body,
        grid=(_GRID,),
        in_specs=[_p_spec, _degp_spec,
                  pl.BlockSpec((H, C), lambda i: (0, 0)),
                  pl.BlockSpec((1, C), lambda i: (0, 0))],
        out_specs=pl.BlockSpec((128, C), lambda i: (i, 0)),
        out_shape=jax.ShapeDtypeStruct((N, C), jnp.float32),
    )


def kernel(features_list, edge_index, e_feat, W_fc, b_fc, b0, W1, b1, W2, b2):
    src = edge_index[0]
    dst = edge_index[1]
    padw = EPW - E // NW
    tr = jnp.arange(padw, dtype=jnp.int32)
    t_dst = jnp.broadcast_to(N + tr % (NPAD - N), (NW, padw))
    t_src_g = jnp.broadcast_to(tr % N, (NW, padw))
    t_src_d = t_dst
    src_g = jnp.concatenate([src.reshape(NW, E // NW), t_src_g], axis=1)
    src_d = jnp.concatenate([src.reshape(NW, E // NW), t_src_d], axis=1)
    dst_p = jnp.concatenate([dst.reshape(NW, E // NW), t_dst], axis=1)
    pk_g3 = ((src_g << 14) | dst_p).reshape(NW, NCHUNK, CHUNK)
    pk_d3 = ((src_d << 14) | dst_p).reshape(NW, NCHUNK, CHUNK)

    zero1 = jnp.zeros((NPAD,), jnp.float32)
    zero2 = jnp.zeros((NPAD, H), jnp.float32)

    degp = _deg_call(pk_d3, zero1)
    degp4 = degp.reshape(NC, 2, NPAD, 1)

    b_fc2 = b_fc.reshape(1, H)
    b0_2 = b0.reshape(1, H)
    b1_2 = b1.reshape(1, H)
    b2_2 = b2.reshape(1, C)

    hs0 = _mk_fc()(features_list, W_fc, b_fc2, degp4)
    p1 = _agg_call(hs0, pk_g3, zero2)
    hs1 = _mk_layer_a()(p1, degp4, b0_2)
    p2 = _agg_call(hs1, pk_g3, zero2)
    h2, hs2 = _mk_layer_b()(p2, degp4, W1, b1_2)
    p3 = _agg_call(hs2, pk_g3, zero2)
    out = _mk_layer_c()(p3, degp4, W2, b2_2)
    return (out, h2)
